# baseline (speedup 1.0000x reference)
import functools

import jax
import jax.numpy as jnp
from jax import lax
from jax.experimental import pallas as pl
from jax.experimental.pallas import tpu as pltpu
from jax.experimental.pallas import tpu_sc as plsc

N_E = 8192
CODE_DIM = 32
TQ = 512


def _proj_body(emb_ref, pw_ref, pbr_ref, pbc_ref, e_ref, etm2_ref, r2_ref):
    emb = emb_ref[...]
    pw = pw_ref[...]
    e = lax.dot_general(emb, pw, (((1,), (1,)), ((), ())),
                        preferred_element_type=jnp.float32) + pbr_ref[...]
    e_ref[...] = e
    et = lax.dot_general(pw, emb, (((1,), (1,)), ((), ())),
                         preferred_element_type=jnp.float32) + pbc_ref[...]
    etm2_ref[...] = -2.0 * et
    r2_ref[...] = jnp.sum(et * et, axis=0, keepdims=True)


def _project(embedding, proj_w, proj_b):
    pbr = proj_b.reshape(1, CODE_DIM)
    pbc = proj_b.reshape(CODE_DIM, 1)
    return pl.pallas_call(
        _proj_body,
        out_shape=(
            jax.ShapeDtypeStruct((N_E, CODE_DIM), jnp.float32),
            jax.ShapeDtypeStruct((CODE_DIM, N_E), jnp.float32),
            jax.ShapeDtypeStruct((1, N_E), jnp.float32),
        ),
    )(embedding, proj_w, pbr, pbc)


def _dist_body(x_ref, etm2_ref, r2_ref, cols_ref, idx_ref, logs_ref):
    q = x_ref[...]
    qe2 = lax.dot_general(q, etm2_ref[...], (((1,), (0,)), ((), ())),
                          preferred_element_type=jnp.float32)
    q2 = jnp.sum(q * q, axis=1, keepdims=True)
    sq = (q2 + r2_ref[...]) + qe2
    m = jnp.min(sq, axis=1, keepdims=True)

    c2 = 2.0813689810056077
    vc = jnp.maximum(sq * c2, 1e-37)
    dd = vc * lax.rsqrt(vc)
    s_raw = jnp.sum(jnp.exp2(-dd), axis=1, keepdims=True)

    dmin = jnp.sqrt(jnp.maximum(m, 0.0))
    logs_ref[...] = jnp.log(s_raw) + dmin

    thrw = jnp.maximum(m, 0.0) * (1.0 + 1e-6)
    mask = (sq <= thrw).astype(jnp.bfloat16)
    agg = lax.dot_general(mask, cols_ref[...], (((1,), (0,)), ((), ())),
                          preferred_element_type=jnp.float32)
    cnt = agg[:, 0:1]
    idx_f = 64.0 * agg[:, 1:2] + agg[:, 2:3]
    multi = jnp.max(cnt) > 1.5

    @pl.when(jnp.logical_not(multi))
    def _fast():
        idx_ref[...] = idx_f.astype(jnp.int32)

    @pl.when(multi)
    def _exact():
        d = jnp.sqrt(jnp.maximum(sq, 0.0))
        iota = lax.broadcasted_iota(jnp.int32, (TQ, N_E), 1)
        idx_ref[...] = jnp.min(jnp.where(d == dmin, iota, N_E), axis=1,
                               keepdims=True)


def _distances(xf, et, r2):
    n = xf.shape[0]
    grid = (n // TQ,)
    j = jnp.arange(N_E, dtype=jnp.int32)
    cols = jnp.stack(
        [jnp.ones((N_E,), jnp.float32),
         (j // 64).astype(jnp.float32),
         (j % 64).astype(jnp.float32)], axis=1).astype(jnp.bfloat16)
    return pl.pallas_call(
        _dist_body,
        grid=grid,
        in_specs=[
            pl.BlockSpec((TQ, CODE_DIM), lambda i: (i, 0)),
            pl.BlockSpec((CODE_DIM, N_E), lambda i: (0, 0)),
            pl.BlockSpec((1, N_E), lambda i: (0, 0)),
            pl.BlockSpec((N_E, 3), lambda i: (0, 0)),
        ],
        out_specs=(
            pl.BlockSpec((TQ, 1), lambda i: (i, 0)),
            pl.BlockSpec((TQ, 1), lambda i: (i, 0)),
        ),
        out_shape=(
            jax.ShapeDtypeStruct((n, 1), jnp.int32),
            jax.ShapeDtypeStruct((n, 1), jnp.float32),
        ),
    )(xf, et, r2, cols)


def _sc_gather(table, idx):
    info = plsc.get_sparse_core_info()
    nc, ns = info.num_cores, info.num_subcores
    nw = nc * ns
    b = idx.shape[0]
    bpw = b // nw
    chunk = 128
    mesh = plsc.VectorSubcoreMesh(core_axis_name="c", subcore_axis_name="s")

    @functools.partial(
        pl.kernel,
        mesh=mesh,
        compiler_params=pltpu.CompilerParams(use_tc_tiling_on_sc=False),
        out_type=jax.ShapeDtypeStruct((b, CODE_DIM), jnp.float32),
        scratch_types=[
            pltpu.VMEM((bpw,), jnp.int32),
            pltpu.VMEM((bpw, CODE_DIM), jnp.float32),
            pltpu.SemaphoreType.DMA,
        ],
    )
    def k(table_hbm, idx_hbm, out_hbm, idx_v, rows_v, sem):
        wid = lax.axis_index("s") * nc + lax.axis_index("c")
        base = wid * bpw
        pltpu.sync_copy(idx_hbm.at[pl.ds(base, bpw)], idx_v)
        for j in range(bpw // chunk):
            pltpu.async_copy(
                table_hbm.at[idx_v.at[pl.ds(j * chunk, chunk)]],
                rows_v.at[pl.ds(j * chunk, chunk)],
                sem,
            ).wait()
        pltpu.sync_copy(rows_v, out_hbm.at[pl.ds(base, bpw)])

    return k(table, idx)


def kernel(x, embedding, proj_w, proj_b):
    bb, tt, cc = x.shape
    xf = x.astype(jnp.float32).reshape(-1, cc)
    e, et, r2 = _project(embedding, proj_w, proj_b)
    idx2, logs2 = _distances(xf, et, r2)
    x_q = _sc_gather(e, idx2.reshape(-1))
    loss = jnp.mean(logs2)
    return x_q.reshape(bb, tt, cc), loss, idx2.reshape(bb, tt, 1)


# speedup vs baseline: 1.0742x; 1.0742x over previous
import functools

import jax
import jax.numpy as jnp
from jax import lax
from jax.experimental import pallas as pl
from jax.experimental.pallas import tpu as pltpu
from jax.experimental.pallas import tpu_sc as plsc

N_E = 8192
CODE_DIM = 32
TQ = 256


def _proj_body(emb_ref, pw_ref, pbr_ref, pbc_ref, e_ref, etm2_ref, r2_ref):
    emb = emb_ref[...]
    pw = pw_ref[...]
    e = lax.dot_general(emb, pw, (((1,), (1,)), ((), ())),
                        preferred_element_type=jnp.float32) + pbr_ref[...]
    e_ref[...] = e
    et = lax.dot_general(pw, emb, (((1,), (1,)), ((), ())),
                         preferred_element_type=jnp.float32) + pbc_ref[...]
    etm2_ref[...] = -2.0 * et
    r2_ref[...] = jnp.sum(et * et, axis=0, keepdims=True)


def _project(embedding, proj_w, proj_b):
    pbr = proj_b.reshape(1, CODE_DIM)
    pbc = proj_b.reshape(CODE_DIM, 1)
    return pl.pallas_call(
        _proj_body,
        out_shape=(
            jax.ShapeDtypeStruct((N_E, CODE_DIM), jnp.float32),
            jax.ShapeDtypeStruct((CODE_DIM, N_E), jnp.float32),
            jax.ShapeDtypeStruct((1, N_E), jnp.float32),
        ),
    )(embedding, proj_w, pbr, pbc)


def _dist_body(x_ref, etm2_ref, r2_ref, cols_ref, idx_ref, logs_ref):
    q = x_ref[...]
    qe2 = lax.dot_general(q, etm2_ref[...], (((1,), (0,)), ((), ())),
                          preferred_element_type=jnp.float32)
    q2 = jnp.sum(q * q, axis=1, keepdims=True)
    sq = (q2 + r2_ref[...]) + qe2
    m = jnp.min(sq, axis=1, keepdims=True)

    c2 = 2.0813689810056077
    vc = jnp.maximum(sq * c2, 1e-37)
    dd = vc * lax.rsqrt(vc)
    s_raw = jnp.sum(jnp.exp2(-dd), axis=1, keepdims=True)

    dmin = jnp.sqrt(jnp.maximum(m, 0.0))
    logs_ref[...] = jnp.log(s_raw) + dmin

    thrw = jnp.maximum(m, 0.0) * (1.0 + 1e-6)
    mask = (sq <= thrw).astype(jnp.bfloat16)
    agg = lax.dot_general(mask, cols_ref[...], (((1,), (0,)), ((), ())),
                          preferred_element_type=jnp.float32)
    cnt = agg[:, 0:1]
    idx_f = 64.0 * agg[:, 1:2] + agg[:, 2:3]
    multi = jnp.max(cnt) > 1.5

    @pl.when(jnp.logical_not(multi))
    def _fast():
        idx_ref[...] = idx_f.astype(jnp.int32)

    @pl.when(multi)
    def _exact():
        d = jnp.sqrt(jnp.maximum(sq, 0.0))
        iota = lax.broadcasted_iota(jnp.int32, (TQ, N_E), 1)
        idx_ref[...] = jnp.min(jnp.where(d == dmin, iota, N_E), axis=1,
                               keepdims=True)


def _distances(xf, et, r2):
    n = xf.shape[0]
    grid = (n // TQ,)
    j = jnp.arange(N_E, dtype=jnp.int32)
    cols = jnp.stack(
        [jnp.ones((N_E,), jnp.float32),
         (j // 64).astype(jnp.float32),
         (j % 64).astype(jnp.float32)], axis=1).astype(jnp.bfloat16)
    return pl.pallas_call(
        _dist_body,
        grid=grid,
        in_specs=[
            pl.BlockSpec((TQ, CODE_DIM), lambda i: (i, 0)),
            pl.BlockSpec((CODE_DIM, N_E), lambda i: (0, 0)),
            pl.BlockSpec((1, N_E), lambda i: (0, 0)),
            pl.BlockSpec((N_E, 3), lambda i: (0, 0)),
        ],
        out_specs=(
            pl.BlockSpec((TQ, 1), lambda i: (i, 0)),
            pl.BlockSpec((TQ, 1), lambda i: (i, 0)),
        ),
        out_shape=(
            jax.ShapeDtypeStruct((n, 1), jnp.int32),
            jax.ShapeDtypeStruct((n, 1), jnp.float32),
        ),
    )(xf, et, r2, cols)


def _sc_gather(table, idx):
    info = plsc.get_sparse_core_info()
    nc, ns = info.num_cores, info.num_subcores
    nw = nc * ns
    b = idx.shape[0]
    bpw = b // nw
    chunk = 128
    mesh = plsc.VectorSubcoreMesh(core_axis_name="c", subcore_axis_name="s")

    @functools.partial(
        pl.kernel,
        mesh=mesh,
        compiler_params=pltpu.CompilerParams(use_tc_tiling_on_sc=False),
        out_type=jax.ShapeDtypeStruct((b, CODE_DIM), jnp.float32),
        scratch_types=[
            pltpu.VMEM((bpw,), jnp.int32),
            pltpu.VMEM((bpw, CODE_DIM), jnp.float32),
            pltpu.SemaphoreType.DMA,
        ],
    )
    def k(table_hbm, idx_hbm, out_hbm, idx_v, rows_v, sem):
        wid = lax.axis_index("s") * nc + lax.axis_index("c")
        base = wid * bpw
        pltpu.sync_copy(idx_hbm.at[pl.ds(base, bpw)], idx_v)
        for j in range(bpw // chunk):
            pltpu.async_copy(
                table_hbm.at[idx_v.at[pl.ds(j * chunk, chunk)]],
                rows_v.at[pl.ds(j * chunk, chunk)],
                sem,
            ).wait()
        pltpu.sync_copy(rows_v, out_hbm.at[pl.ds(base, bpw)])

    return k(table, idx)


def kernel(x, embedding, proj_w, proj_b):
    bb, tt, cc = x.shape
    xf = x.astype(jnp.float32).reshape(-1, cc)
    e, et, r2 = _project(embedding, proj_w, proj_b)
    idx2, logs2 = _distances(xf, et, r2)
    x_q = _sc_gather(e, idx2.reshape(-1))
    loss = jnp.mean(logs2)
    return x_q.reshape(bb, tt, cc), loss, idx2.reshape(bb, tt, 1)
